# Initial kernel scaffold; baseline (speedup 1.0000x reference)
import jax
import jax.numpy as jnp
from jax.experimental import pallas as pl


def kernel(x, edge_index, edge_attr, W_edge, t, W1, b1, W2, b2):
    raise NotImplementedError("write your pallas kernel here")


# speedup vs baseline: 2.1806x; 2.1806x over previous
import functools

import jax
import jax.numpy as jnp
from jax import lax
from jax.experimental import pallas as pl
from jax.experimental.pallas import tpu as pltpu
from jax.experimental.pallas import tpu_sc as plsc

N = 10000
E = 320000
D = 128
HALF = 64
EPS = 1e-7

NSUB = 16
EDGES_PER_SUB = E // NSUB
CHUNK = 80
NCHUNK = EDGES_PER_SUB // CHUNK
ROWS_PER_SUB = N // NSUB
FIN_CHUNK = 125
NFIN = ROWS_PER_SUB // FIN_CHUNK


def _edge_encode_body(attr_ref, w_ref, out_ref):
    out_ref[...] = jnp.dot(attr_ref[...], w_ref[0],
                           preferred_element_type=jnp.float32)[None]


def _edge_encode(edge_attr, W_edge):
    EB = 2000
    w_halves = jnp.stack([W_edge[:, :HALF], W_edge[:, HALF:]])
    return pl.pallas_call(
        _edge_encode_body,
        grid=(2, E // EB),
        in_specs=[
            pl.BlockSpec((EB, 16), lambda h, i: (i, 0)),
            pl.BlockSpec((1, 16, HALF), lambda h, i: (h, 0, 0)),
        ],
        out_specs=pl.BlockSpec((1, EB, HALF), lambda h, i: (h, i, 0)),
        out_shape=jax.ShapeDtypeStruct((2, E, HALF), jnp.float32),
    )(edge_attr, w_halves)


def _sc_aggregate(x2, e2, srcadj, dst, t, zeros):
    mesh = plsc.VectorSubcoreMesh(core_axis_name="c", subcore_axis_name="s")

    @functools.partial(
        pl.kernel,
        out_type=jax.ShapeDtypeStruct((N, D), jnp.float32),
        mesh=mesh,
        compiler_params=pltpu.CompilerParams(use_tc_tiling_on_sc=False),
        scratch_types=[
            pltpu.VMEM((CHUNK,), jnp.int32),
            pltpu.VMEM((CHUNK,), jnp.int32),
            pltpu.VMEM((CHUNK, HALF), jnp.float32),
            pltpu.VMEM((CHUNK, HALF), jnp.float32),
            pltpu.VMEM((HALF,), jnp.float32),
            pltpu.VMEM((FIN_CHUNK, HALF), jnp.float32),
            pltpu.VMEM((FIN_CHUNK, HALF), jnp.float32),
            pltpu.VMEM_SHARED((N, HALF), jnp.float32),
            pltpu.VMEM_SHARED((N, HALF), jnp.float32),
            pltpu.SemaphoreType.DMA,
        ],
    )
    def body(x2_hbm, e2_hbm, srcadj_hbm, dst_hbm, t_hbm, z_hbm, out_hbm,
             sidx, didx, xbuf, ebuf, tbuf, nbuf, dbuf, num_sh, den_sh, sem):
        c = lax.axis_index("c")
        s = lax.axis_index("s")
        row0 = s * ROWS_PER_SUB
        rows = pl.ds(row0, ROWS_PER_SUB)
        pltpu.sync_copy(z_hbm.at[rows, :], num_sh.at[rows, :])
        pltpu.sync_copy(z_hbm.at[rows, :], den_sh.at[rows, :])
        pltpu.sync_copy(t_hbm.at[pl.ds(c * HALF, HALF)], tbuf)
        plsc.subcore_barrier()

        ebase = s * EDGES_PER_SUB

        def chunk_body(i, carry):
            base = ebase + i * CHUNK
            pltpu.sync_copy(srcadj_hbm.at[c, pl.ds(base, CHUNK)], sidx)
            gather = pltpu.async_copy(x2_hbm.at[sidx], xbuf, sem)
            pltpu.sync_copy(e2_hbm.at[c, pl.ds(base, CHUNK), :], ebuf)
            pltpu.sync_copy(dst_hbm.at[pl.ds(base, CHUNK)], didx)
            gather.wait()

            def row_body(r, rcarry):
                for j in range(HALF // 16):
                    sl = pl.ds(j * 16, 16)
                    xv = xbuf[r, sl]
                    ev = ebuf[r, sl]
                    m = jnp.maximum(xv + ev, 0.0) + EPS
                    ex = jnp.exp(m * tbuf[sl])
                    ebuf[r, sl] = ex
                    xbuf[r, sl] = ex * m
                return rcarry

            lax.fori_loop(0, CHUNK, row_body, 0)
            pltpu.sync_copy(xbuf, num_sh.at[didx], add=True)
            pltpu.sync_copy(ebuf, den_sh.at[didx], add=True)
            return carry

        lax.fori_loop(0, NCHUNK, chunk_body, 0)
        plsc.subcore_barrier()

        def fin_body(k, carry):
            r0 = row0 + k * FIN_CHUNK
            pltpu.sync_copy(num_sh.at[pl.ds(r0, FIN_CHUNK), :], nbuf)
            pltpu.sync_copy(den_sh.at[pl.ds(r0, FIN_CHUNK), :], dbuf)

            def rb(r, rcarry):
                for j in range(HALF // 16):
                    sl = pl.ds(j * 16, 16)
                    nbuf[r, sl] = nbuf[r, sl] / (dbuf[r, sl] + 1e-16)
                return rcarry

            lax.fori_loop(0, FIN_CHUNK, rb, 0)
            pltpu.sync_copy(
                nbuf, out_hbm.at[pl.ds(r0, FIN_CHUNK), pl.ds(c * HALF, HALF)])
            return carry

        lax.fori_loop(0, NFIN, fin_body, 0)

    return body(x2, e2, srcadj, dst, t, zeros)


def _mlp_body(x_ref, a_ref, w1_ref, b1_ref, w2_ref, b2_ref, o_ref):
    h = x_ref[...] + a_ref[...]
    h1 = jnp.maximum(
        jnp.dot(h, w1_ref[...], preferred_element_type=jnp.float32)
        + b1_ref[...], 0.0)
    h2 = jnp.dot(h1, w2_ref[...], preferred_element_type=jnp.float32) \
        + b2_ref[...]
    o_ref[...] = x_ref[...] + jnp.maximum(h2, 0.0)


def _mlp(x, aggr, W1, b1, W2, b2):
    RB = 2000
    return pl.pallas_call(
        _mlp_body,
        grid=(N // RB,),
        in_specs=[
            pl.BlockSpec((RB, D), lambda i: (i, 0)),
            pl.BlockSpec((RB, D), lambda i: (i, 0)),
            pl.BlockSpec((D, 2 * D), lambda i: (0, 0)),
            pl.BlockSpec((1, 2 * D), lambda i: (0, 0)),
            pl.BlockSpec((2 * D, D), lambda i: (0, 0)),
            pl.BlockSpec((1, D), lambda i: (0, 0)),
        ],
        out_specs=pl.BlockSpec((RB, D), lambda i: (i, 0)),
        out_shape=jax.ShapeDtypeStruct((N, D), jnp.float32),
    )(x, aggr, W1, b1, W2, b2)


def kernel(x, edge_index, edge_attr, W_edge, t, W1, b1, W2, b2):
    src = edge_index[0].astype(jnp.int32)
    dst = edge_index[1].astype(jnp.int32)
    srcadj = jnp.stack([src, src + N])
    x2 = jnp.concatenate([x[:, :HALF], x[:, HALF:]], axis=0)
    e2 = _edge_encode(edge_attr, W_edge)
    zeros = jnp.zeros((N, HALF), jnp.float32)
    aggr = _sc_aggregate(x2, e2, srcadj, dst, t, zeros)
    return _mlp(x, aggr, W1, b1.reshape(1, -1), W2, b2.reshape(1, -1))


# speedup vs baseline: 4.3140x; 1.9783x over previous
import functools

import jax
import jax.numpy as jnp
from jax import lax
from jax.experimental import pallas as pl
from jax.experimental.pallas import tpu as pltpu
from jax.experimental.pallas import tpu_sc as plsc

N = 10000
E = 320000
D = 128
HALF = 64
EPS = 1e-7

NSUB = 16
EDGES_PER_SUB = E // NSUB
CHUNK = 80
NCHUNK = EDGES_PER_SUB // CHUNK
ROWS_PER_SUB = N // NSUB
FIN_CHUNK = 125
NFIN = ROWS_PER_SUB // FIN_CHUNK


def _edge_encode_body(attr_ref, w_ref, out_ref):
    out_ref[...] = jnp.dot(attr_ref[...], w_ref[0],
                           preferred_element_type=jnp.float32)[None]


def _edge_encode(edge_attr, W_edge):
    EB = 2000
    w_halves = jnp.stack([W_edge[:, :HALF], W_edge[:, HALF:]])
    return pl.pallas_call(
        _edge_encode_body,
        grid=(2, E // EB),
        in_specs=[
            pl.BlockSpec((EB, 16), lambda h, i: (i, 0)),
            pl.BlockSpec((1, 16, HALF), lambda h, i: (h, 0, 0)),
        ],
        out_specs=pl.BlockSpec((1, EB, HALF), lambda h, i: (h, i, 0)),
        out_shape=jax.ShapeDtypeStruct((2, E, HALF), jnp.float32),
    )(edge_attr, w_halves)


def _sc_aggregate(x2, e2, srcadj, dst, t, zeros):
    mesh = plsc.VectorSubcoreMesh(core_axis_name="c", subcore_axis_name="s")

    @functools.partial(
        pl.kernel,
        out_type=jax.ShapeDtypeStruct((N, D), jnp.float32),
        mesh=mesh,
        compiler_params=pltpu.CompilerParams(use_tc_tiling_on_sc=False),
        scratch_types=[
            pltpu.VMEM((2, CHUNK), jnp.int32),
            pltpu.VMEM((2, CHUNK), jnp.int32),
            pltpu.VMEM((2, CHUNK, HALF), jnp.float32),
            pltpu.VMEM((2, CHUNK, HALF), jnp.float32),
            pltpu.VMEM((HALF,), jnp.float32),
            pltpu.VMEM((FIN_CHUNK, HALF), jnp.float32),
            pltpu.VMEM((FIN_CHUNK, HALF), jnp.float32),
            pltpu.VMEM_SHARED((N, HALF), jnp.float32),
            pltpu.VMEM_SHARED((N, HALF), jnp.float32),
            [pltpu.SemaphoreType.DMA] * 2,
            [pltpu.SemaphoreType.DMA] * 2,
        ],
    )
    def body(x2_hbm, e2_hbm, srcadj_hbm, dst_hbm, t_hbm, z_hbm, out_hbm,
             sidx, didx, xbuf, ebuf, tbuf, nbuf, dbuf, num_sh, den_sh,
             dsem, isem):
        c = lax.axis_index("c")
        s = lax.axis_index("s")
        row0 = s * ROWS_PER_SUB
        rows = pl.ds(row0, ROWS_PER_SUB)
        pltpu.sync_copy(z_hbm.at[rows, :], num_sh.at[rows, :])
        pltpu.sync_copy(z_hbm.at[rows, :], den_sh.at[rows, :])
        pltpu.sync_copy(t_hbm.at[pl.ds(c * HALF, HALF)], tbuf)
        plsc.subcore_barrier()

        ebase = s * EDGES_PER_SUB
        LAST = NCHUNK - 1

        def issue_idx(i, b):
            base = ebase + i * CHUNK
            pltpu.async_copy(srcadj_hbm.at[c, pl.ds(base, CHUNK)],
                             sidx.at[b], isem[b])
            pltpu.async_copy(dst_hbm.at[pl.ds(base, CHUNK)],
                             didx.at[b], isem[b])

        def issue_data(i, b):
            base = ebase + i * CHUNK
            pltpu.async_copy(x2_hbm.at[sidx.at[b]], xbuf.at[b], dsem[b])
            pltpu.async_copy(e2_hbm.at[c, pl.ds(base, CHUNK), :],
                             ebuf.at[b], dsem[b])

        def wait_idx(b):
            pltpu.make_async_copy(dst_hbm.at[pl.ds(0, CHUNK)],
                                  sidx.at[b], isem[b]).wait()
            pltpu.make_async_copy(dst_hbm.at[pl.ds(0, CHUNK)],
                                  didx.at[b], isem[b]).wait()

        def wait_data(b):
            pltpu.make_async_copy(x2_hbm.at[pl.ds(0, CHUNK), :],
                                  xbuf.at[b], dsem[b]).wait()
            pltpu.make_async_copy(x2_hbm.at[pl.ds(0, CHUNK), :],
                                  ebuf.at[b], dsem[b]).wait()

        def stage(i, b, nb):
            wait_idx(nb)
            issue_data(jnp.minimum(i + 1, LAST), nb)
            wait_data(b)
            xb = xbuf.at[b]
            eb = ebuf.at[b]

            @plsc.parallel_loop(0, CHUNK, unroll=8)
            def row_body(r):
                for j in range(HALF // 16):
                    sl = pl.ds(j * 16, 16)
                    xv = xb[r, sl]
                    ev = eb[r, sl]
                    m = jnp.maximum(xv + ev, 0.0) + EPS
                    ex = jnp.exp(m * tbuf[sl])
                    eb[r, sl] = ex
                    xb[r, sl] = ex * m

            pltpu.sync_copy(xb, num_sh.at[didx.at[b]], add=True)
            pltpu.sync_copy(eb, den_sh.at[didx.at[b]], add=True)
            issue_idx(jnp.minimum(i + 2, LAST), b)

        issue_idx(0, 0)
        wait_idx(0)
        issue_data(0, 0)
        issue_idx(jnp.minimum(1, LAST), 1)

        def pair_body(k, carry):
            i = 2 * k
            stage(i, 0, 1)
            stage(i + 1, 1, 0)
            return carry

        lax.fori_loop(0, NCHUNK // 2, pair_body, 0)
        wait_data(0)
        wait_idx(1)
        plsc.subcore_barrier()

        def fin_body(k, carry):
            r0 = row0 + k * FIN_CHUNK
            pltpu.sync_copy(num_sh.at[pl.ds(r0, FIN_CHUNK), :], nbuf)
            pltpu.sync_copy(den_sh.at[pl.ds(r0, FIN_CHUNK), :], dbuf)

            def rb(r, rcarry):
                for j in range(HALF // 16):
                    sl = pl.ds(j * 16, 16)
                    nbuf[r, sl] = nbuf[r, sl] / (dbuf[r, sl] + 1e-16)
                return rcarry

            lax.fori_loop(0, FIN_CHUNK, rb, 0)
            pltpu.sync_copy(
                nbuf, out_hbm.at[pl.ds(r0, FIN_CHUNK), pl.ds(c * HALF, HALF)])
            return carry

        lax.fori_loop(0, NFIN, fin_body, 0)

    return body(x2, e2, srcadj, dst, t, zeros)


def _mlp_body(x_ref, a_ref, w1_ref, b1_ref, w2_ref, b2_ref, o_ref):
    h = x_ref[...] + a_ref[...]
    h1 = jnp.maximum(
        jnp.dot(h, w1_ref[...], preferred_element_type=jnp.float32)
        + b1_ref[...], 0.0)
    h2 = jnp.dot(h1, w2_ref[...], preferred_element_type=jnp.float32) \
        + b2_ref[...]
    o_ref[...] = x_ref[...] + jnp.maximum(h2, 0.0)


def _mlp(x, aggr, W1, b1, W2, b2):
    RB = 2000
    return pl.pallas_call(
        _mlp_body,
        grid=(N // RB,),
        in_specs=[
            pl.BlockSpec((RB, D), lambda i: (i, 0)),
            pl.BlockSpec((RB, D), lambda i: (i, 0)),
            pl.BlockSpec((D, 2 * D), lambda i: (0, 0)),
            pl.BlockSpec((1, 2 * D), lambda i: (0, 0)),
            pl.BlockSpec((2 * D, D), lambda i: (0, 0)),
            pl.BlockSpec((1, D), lambda i: (0, 0)),
        ],
        out_specs=pl.BlockSpec((RB, D), lambda i: (i, 0)),
        out_shape=jax.ShapeDtypeStruct((N, D), jnp.float32),
    )(x, aggr, W1, b1, W2, b2)


def kernel(x, edge_index, edge_attr, W_edge, t, W1, b1, W2, b2):
    src = edge_index[0].astype(jnp.int32)
    dst = edge_index[1].astype(jnp.int32)
    srcadj = jnp.stack([src, src + N])
    x2 = jnp.concatenate([x[:, :HALF], x[:, HALF:]], axis=0)
    e2 = _edge_encode(edge_attr, W_edge)
    zeros = jnp.zeros((N, HALF), jnp.float32)
    aggr = _sc_aggregate(x2, e2, srcadj, dst, t, zeros)
    return _mlp(x, aggr, W1, b1.reshape(1, -1), W2, b2.reshape(1, -1))


# speedup vs baseline: 4.8177x; 1.1167x over previous
import functools

import jax
import jax.numpy as jnp
from jax import lax
from jax.experimental import pallas as pl
from jax.experimental.pallas import tpu as pltpu
from jax.experimental.pallas import tpu_sc as plsc

N = 10000
E = 320000
D = 128
HALF = 64
EPS = 1e-7

NSUB = 16
EDGES_PER_SUB = E // NSUB
CHUNK = 80
NCHUNK = EDGES_PER_SUB // CHUNK
ROWS_PER_SUB = N // NSUB
FIN_CHUNK = 125
NFIN = ROWS_PER_SUB // FIN_CHUNK


def _edge_encode_body(attr_ref, w_ref, out_ref):
    out_ref[...] = jnp.dot(attr_ref[...], w_ref[0],
                           preferred_element_type=jnp.float32)[None]


def _edge_encode(edge_attr, W_edge):
    EB = 1000
    attr2 = edge_attr.reshape(E // 2, 32)
    eye2 = jnp.eye(2, dtype=jnp.float32)
    w_bd = jnp.stack([jnp.kron(eye2, W_edge[:, :HALF]),
                      jnp.kron(eye2, W_edge[:, HALF:])])
    return pl.pallas_call(
        _edge_encode_body,
        grid=(2, (E // 2) // EB),
        in_specs=[
            pl.BlockSpec((EB, 32), lambda h, i: (i, 0)),
            pl.BlockSpec((1, 32, D), lambda h, i: (h, 0, 0)),
        ],
        out_specs=pl.BlockSpec((1, EB, D), lambda h, i: (h, i, 0)),
        out_shape=jax.ShapeDtypeStruct((2, E // 2, D), jnp.float32),
    )(attr2, w_bd)


def _sc_aggregate(x2, e2, srcadj, dst, t, zeros):
    mesh = plsc.VectorSubcoreMesh(core_axis_name="c", subcore_axis_name="s")

    @functools.partial(
        pl.kernel,
        out_type=jax.ShapeDtypeStruct((N, D), jnp.float32),
        mesh=mesh,
        compiler_params=pltpu.CompilerParams(use_tc_tiling_on_sc=False),
        scratch_types=[
            pltpu.VMEM((2, CHUNK), jnp.int32),
            pltpu.VMEM((2, CHUNK), jnp.int32),
            pltpu.VMEM((2, CHUNK, HALF), jnp.float32),
            pltpu.VMEM((2, CHUNK // 2, D), jnp.float32),
            pltpu.VMEM((2, CHUNK, HALF), jnp.float32),
            pltpu.VMEM((HALF,), jnp.float32),
            pltpu.VMEM((FIN_CHUNK, HALF), jnp.float32),
            pltpu.VMEM((FIN_CHUNK, HALF), jnp.float32),
            pltpu.VMEM_SHARED((N, HALF), jnp.float32),
            pltpu.VMEM_SHARED((N, HALF), jnp.float32),
            [pltpu.SemaphoreType.DMA] * 2,
            [pltpu.SemaphoreType.DMA] * 2,
        ],
    )
    def body(x2_hbm, e2_hbm, srcadj_hbm, dst_hbm, t_hbm, out_hbm,
             sidx, didx, xbuf, ebuf, exb, tbuf, nbuf, dbuf, num_sh, den_sh,
             dsem, isem):
        c = lax.axis_index("c")
        s = lax.axis_index("s")
        row0 = s * ROWS_PER_SUB

        def zero_rb(r, rcarry):
            for j in range(HALF // 16):
                nbuf[r, pl.ds(j * 16, 16)] = jnp.zeros((16,), jnp.float32)
            return rcarry

        lax.fori_loop(0, FIN_CHUNK, zero_rb, 0)
        for k in range(NFIN):
            zrows = pl.ds(row0 + k * FIN_CHUNK, FIN_CHUNK)
            pltpu.sync_copy(nbuf, num_sh.at[zrows, :])
            pltpu.sync_copy(nbuf, den_sh.at[zrows, :])
        pltpu.sync_copy(t_hbm.at[pl.ds(c * HALF, HALF)], tbuf)
        plsc.subcore_barrier()

        ebase = s * EDGES_PER_SUB
        LAST = NCHUNK - 1

        def issue_idx(i, b):
            base = ebase + i * CHUNK
            pltpu.async_copy(srcadj_hbm.at[c, pl.ds(base, CHUNK)],
                             sidx.at[b], isem[b])
            pltpu.async_copy(dst_hbm.at[pl.ds(base, CHUNK)],
                             didx.at[b], isem[b])

        def issue_data(i, b):
            pbase = s * (EDGES_PER_SUB // 2) + i * (CHUNK // 2)
            pltpu.async_copy(x2_hbm.at[sidx.at[b]], xbuf.at[b], dsem[b])
            pltpu.async_copy(e2_hbm.at[c, pl.ds(pbase, CHUNK // 2), :],
                             ebuf.at[b], dsem[b])

        def wait_idx(b):
            pltpu.make_async_copy(dst_hbm.at[pl.ds(0, CHUNK)],
                                  sidx.at[b], isem[b]).wait()
            pltpu.make_async_copy(dst_hbm.at[pl.ds(0, CHUNK)],
                                  didx.at[b], isem[b]).wait()

        def wait_data(b):
            pltpu.make_async_copy(x2_hbm.at[pl.ds(0, CHUNK), :],
                                  xbuf.at[b], dsem[b]).wait()
            pltpu.make_async_copy(e2_hbm.at[0, pl.ds(0, CHUNK // 2), :],
                                  ebuf.at[b], dsem[b]).wait()

        def stage(i, b, nb):
            wait_idx(nb)
            issue_data(jnp.minimum(i + 1, LAST), nb)
            wait_data(b)
            xb = xbuf.at[b]
            eb = ebuf.at[b]
            xx = exb.at[b]

            @plsc.parallel_loop(0, CHUNK // 2, unroll=4)
            def row_body(pr):
                for k in range(2):
                    r = 2 * pr + k
                    for j in range(HALF // 16):
                        sl = pl.ds(j * 16, 16)
                        xv = xb[r, sl]
                        ev = eb[pr, pl.ds(k * HALF + j * 16, 16)]
                        m = jnp.maximum(xv + ev, 0.0) + EPS
                        ex = jnp.exp(m * tbuf[sl])
                        xx[r, sl] = ex
                        xb[r, sl] = ex * m

            pltpu.sync_copy(xb, num_sh.at[didx.at[b]], add=True)
            pltpu.sync_copy(xx, den_sh.at[didx.at[b]], add=True)
            issue_idx(jnp.minimum(i + 2, LAST), b)

        issue_idx(0, 0)
        wait_idx(0)
        issue_data(0, 0)
        issue_idx(jnp.minimum(1, LAST), 1)

        def pair_body(k, carry):
            i = 2 * k
            stage(i, 0, 1)
            stage(i + 1, 1, 0)
            return carry

        lax.fori_loop(0, NCHUNK // 2, pair_body, 0)
        wait_data(0)
        wait_idx(1)
        plsc.subcore_barrier()

        def fin_body(k, carry):
            r0 = row0 + k * FIN_CHUNK
            pltpu.sync_copy(num_sh.at[pl.ds(r0, FIN_CHUNK), :], nbuf)
            pltpu.sync_copy(den_sh.at[pl.ds(r0, FIN_CHUNK), :], dbuf)

            def rb(r, rcarry):
                for j in range(HALF // 16):
                    sl = pl.ds(j * 16, 16)
                    nbuf[r, sl] = nbuf[r, sl] / (dbuf[r, sl] + 1e-16)
                return rcarry

            lax.fori_loop(0, FIN_CHUNK, rb, 0)
            pltpu.sync_copy(
                nbuf, out_hbm.at[pl.ds(r0, FIN_CHUNK), pl.ds(c * HALF, HALF)])
            return carry

        lax.fori_loop(0, NFIN, fin_body, 0)

    return body(x2, e2, srcadj, dst, t)


def _mlp_body(x_ref, a_ref, w1_ref, b1_ref, w2_ref, b2_ref, o_ref):
    h = x_ref[...] + a_ref[...]
    h1 = jnp.maximum(
        jnp.dot(h, w1_ref[...], preferred_element_type=jnp.float32)
        + b1_ref[...], 0.0)
    h2 = jnp.dot(h1, w2_ref[...], preferred_element_type=jnp.float32) \
        + b2_ref[...]
    o_ref[...] = x_ref[...] + jnp.maximum(h2, 0.0)


def _mlp(x, aggr, W1, b1, W2, b2):
    RB = 2000
    return pl.pallas_call(
        _mlp_body,
        grid=(N // RB,),
        in_specs=[
            pl.BlockSpec((RB, D), lambda i: (i, 0)),
            pl.BlockSpec((RB, D), lambda i: (i, 0)),
            pl.BlockSpec((D, 2 * D), lambda i: (0, 0)),
            pl.BlockSpec((1, 2 * D), lambda i: (0, 0)),
            pl.BlockSpec((2 * D, D), lambda i: (0, 0)),
            pl.BlockSpec((1, D), lambda i: (0, 0)),
        ],
        out_specs=pl.BlockSpec((RB, D), lambda i: (i, 0)),
        out_shape=jax.ShapeDtypeStruct((N, D), jnp.float32),
    )(x, aggr, W1, b1, W2, b2)


def kernel(x, edge_index, edge_attr, W_edge, t, W1, b1, W2, b2):
    src = edge_index[0].astype(jnp.int32)
    dst = edge_index[1].astype(jnp.int32)
    srcadj = jnp.stack([src, src + N])
    x2 = jnp.concatenate([x[:, :HALF], x[:, HALF:]], axis=0)
    e2 = _edge_encode(edge_attr, W_edge)
    zeros = jnp.zeros((N, HALF), jnp.float32)
    aggr = _sc_aggregate(x2, e2, srcadj, dst, t, zeros)
    return _mlp(x, aggr, W1, b1.reshape(1, -1), W2, b2.reshape(1, -1))
